# baseline (speedup 1.0000x reference)
import functools

import jax
import jax.numpy as jnp
from jax.experimental import pallas as pl
from jax.experimental.pallas import tpu as pltpu

B = 16


def _qproj_body(q_ref, wqn_ref, bqn_ref, bn_ref, wqe_ref, bqe_ref, be_ref,
                qn_ref, qe_ref):
    q = q_ref[...]
    qn_ref[...] = (jnp.dot(q, wqn_ref[...], preferred_element_type=jnp.float32)
                   + bqn_ref[...] + bn_ref[...])
    qe_ref[...] = (jnp.dot(q, wqe_ref[...], preferred_element_type=jnp.float32)
                   + bqe_ref[...] + be_ref[...])


def _att_body(feat_ref, ids_ref, w_ref, qtab_ref, wv_ref, bv_ref, out_ref,
              *, chunk):
    feat = feat_ref[...].astype(jnp.bfloat16)
    ids = ids_ref[...]
    onehot = (ids == jax.lax.broadcasted_iota(jnp.int32, (chunk, B), 1)
              ).astype(jnp.bfloat16)
    qpart = jnp.dot(onehot, qtab_ref[...].astype(jnp.bfloat16),
                    preferred_element_type=jnp.float32)
    proj = jnp.dot(feat, w_ref[...].astype(jnp.bfloat16),
                   preferred_element_type=jnp.float32)
    t = jnp.tanh(qpart + proj).astype(jnp.bfloat16)
    s = (jnp.dot(t, wv_ref[...].astype(jnp.bfloat16),
                 preferred_element_type=jnp.float32) + bv_ref[...])
    m = jnp.max(s, axis=1, keepdims=True)
    e = jnp.exp(s - m)
    out_ref[...] = e / jnp.sum(e, axis=1, keepdims=True)


def _run_branch(feat, ids, w, qtab, wv, bv, chunk):
    n = feat.shape[0]
    d = feat.shape[1]
    p = w.shape[1]
    grid = n // chunk
    body = functools.partial(_att_body, chunk=chunk)
    out = pl.pallas_call(
        body,
        grid=(grid,),
        in_specs=[
            pl.BlockSpec((chunk, d), lambda i: (i, 0)),
            pl.BlockSpec((chunk, 1), lambda i: (i, 0)),
            pl.BlockSpec((d, p), lambda i: (0, 0)),
            pl.BlockSpec((B, p), lambda i: (0, 0)),
            pl.BlockSpec((p, 1), lambda i: (0, 0)),
            pl.BlockSpec((1, 1), lambda i: (0, 0)),
        ],
        out_specs=pl.BlockSpec((chunk, 1), lambda i: (i, 0)),
        out_shape=jax.ShapeDtypeStruct((n, 1), jnp.float32),
        compiler_params=pltpu.CompilerParams(
            dimension_semantics=("parallel",)),
    )(feat, ids, w, qtab, wv, bv)
    return out[:, 0]


def _att_body_t(feat_ref, ids_ref, wcat_ref, wv_ref, bv_ref, out_ref,
                *, chunk):
    feat_t = jnp.transpose(feat_ref[...].astype(jnp.bfloat16))
    ids = ids_ref[...].reshape(1, chunk)
    onehot_t = (ids == jax.lax.broadcasted_iota(jnp.int32, (B, chunk), 0)
                ).astype(jnp.bfloat16)
    a = jnp.concatenate([onehot_t, feat_t], axis=0)
    x = jax.lax.dot_general(
        wcat_ref[...].astype(jnp.bfloat16), a,
        (((0,), (0,)), ((), ())), preferred_element_type=jnp.float32)
    t = jnp.tanh(x).astype(jnp.bfloat16)
    s = jax.lax.dot_general(
        wv_ref[...].astype(jnp.bfloat16), t,
        (((0,), (0,)), ((), ())), preferred_element_type=jnp.float32)
    s = s + bv_ref[...]
    m = jnp.max(s, axis=0, keepdims=True)
    e = jnp.exp(s - m)
    out_ref[...] = (e / jnp.sum(e, axis=0, keepdims=True)).reshape(
        chunk // 128, 128)


def _run_branch_t(feat, ids3, wcat, wv, bv, chunk):
    n = feat.shape[0]
    d = feat.shape[1]
    p = wcat.shape[1]
    grid = n // chunk
    body = functools.partial(_att_body_t, chunk=chunk)
    out = pl.pallas_call(
        body,
        grid=(grid,),
        in_specs=[
            pl.BlockSpec((chunk, d), lambda i: (i, 0)),
            pl.BlockSpec((chunk // 128, 128), lambda i: (i, 0)),
            pl.BlockSpec((B + d, p), lambda i: (0, 0)),
            pl.BlockSpec((p, 1), lambda i: (0, 0)),
            pl.BlockSpec((1, 1), lambda i: (0, 0)),
        ],
        out_specs=pl.BlockSpec((chunk // 128, 128), lambda i: (i, 0)),
        out_shape=jax.ShapeDtypeStruct((n // 128, 128), jnp.float32),
        compiler_params=pltpu.CompilerParams(
            dimension_semantics=("parallel",)),
    )(feat, ids3.reshape(n // 128, 128), wcat, wv, bv)
    return out.reshape(-1)


def kernel(question, node_feat, edge_feat, node_graph_ids, edge_graph_ids,
           Wqn, bqn, Wn, bn, wvn, bvn, Wqe, bqe, We, be, wve, bve):
    qn, qe = pl.pallas_call(
        _qproj_body,
        out_shape=(jax.ShapeDtypeStruct((B, Wqn.shape[1]), jnp.float32),
                   jax.ShapeDtypeStruct((B, Wqe.shape[1]), jnp.float32)),
    )(question, Wqn, bqn.reshape(1, -1), bn.reshape(1, -1),
      Wqe, bqe.reshape(1, -1), be.reshape(1, -1))

    node_ids = node_graph_ids.astype(jnp.int32).reshape(-1, 1)
    edge_ids3 = edge_graph_ids.astype(jnp.int32).reshape(-1, 1, 25600)

    node_att = _run_branch(node_feat, node_ids, Wn, qn,
                           wvn, bvn.reshape(1, 1), chunk=10000)
    edge_att = _run_branch_t(edge_feat, edge_ids3, jnp.concatenate([qe, We]),
                             wve, bve.reshape(1, 1), chunk=25600)
    return (node_att, edge_att)


# speedup vs baseline: 1.0664x; 1.0664x over previous
import functools

import jax
import jax.numpy as jnp
from jax.experimental import pallas as pl
from jax.experimental.pallas import tpu as pltpu

B = 16


def _qproj_body(q_ref, wqn_ref, bqn_ref, bn_ref, wqe_ref, bqe_ref, be_ref,
                qn_ref, qe_ref):
    q = q_ref[...]
    qn_ref[...] = (jnp.dot(q, wqn_ref[...], preferred_element_type=jnp.float32)
                   + bqn_ref[...] + bn_ref[...])
    qe_ref[...] = (jnp.dot(q, wqe_ref[...], preferred_element_type=jnp.float32)
                   + bqe_ref[...] + be_ref[...])


def _att_body(feat_ref, ids_ref, w_ref, qtab_ref, wv_ref, bv_ref, out_ref,
              *, chunk):
    feat = feat_ref[...].astype(jnp.bfloat16)
    ids = ids_ref[...]
    onehot = (ids == jax.lax.broadcasted_iota(jnp.int32, (chunk, B), 1)
              ).astype(jnp.bfloat16)
    qpart = jnp.dot(onehot, qtab_ref[...].astype(jnp.bfloat16),
                    preferred_element_type=jnp.float32)
    proj = jnp.dot(feat, w_ref[...].astype(jnp.bfloat16),
                   preferred_element_type=jnp.float32)
    t = jnp.tanh(qpart + proj).astype(jnp.bfloat16)
    s = (jnp.dot(t, wv_ref[...].astype(jnp.bfloat16),
                 preferred_element_type=jnp.float32) + bv_ref[...])
    m = jnp.max(s, axis=1, keepdims=True)
    e = jnp.exp(s - m)
    out_ref[...] = e / jnp.sum(e, axis=1, keepdims=True)


def _run_branch(feat, ids, w, qtab, wv, bv, chunk):
    n = feat.shape[0]
    d = feat.shape[1]
    p = w.shape[1]
    grid = n // chunk
    body = functools.partial(_att_body, chunk=chunk)
    out = pl.pallas_call(
        body,
        grid=(grid,),
        in_specs=[
            pl.BlockSpec((chunk, d), lambda i: (i, 0)),
            pl.BlockSpec((chunk, 1), lambda i: (i, 0)),
            pl.BlockSpec((d, p), lambda i: (0, 0)),
            pl.BlockSpec((B, p), lambda i: (0, 0)),
            pl.BlockSpec((p, 1), lambda i: (0, 0)),
            pl.BlockSpec((1, 1), lambda i: (0, 0)),
        ],
        out_specs=pl.BlockSpec((chunk, 1), lambda i: (i, 0)),
        out_shape=jax.ShapeDtypeStruct((n, 1), jnp.float32),
        compiler_params=pltpu.CompilerParams(
            dimension_semantics=("parallel",)),
    )(feat, ids, w, qtab, wv, bv)
    return out[:, 0]


def _att_body_t(feat_ref, ids_ref, wcat_ref, wv_ref, bv_ref, out_ref,
                *, chunk):
    feat_t = jnp.transpose(feat_ref[...].astype(jnp.bfloat16))
    ids = ids_ref[...].reshape(1, chunk)
    onehot_t = (ids == jax.lax.broadcasted_iota(jnp.int32, (B, chunk), 0)
                ).astype(jnp.bfloat16)
    a = jnp.concatenate([onehot_t, feat_t], axis=0)
    x = jax.lax.dot_general(
        wcat_ref[...].astype(jnp.bfloat16), a,
        (((0,), (0,)), ((), ())), preferred_element_type=jnp.float32)
    t = jnp.tanh(x).astype(jnp.bfloat16)
    s = jax.lax.dot_general(
        wv_ref[...].astype(jnp.bfloat16), t,
        (((0,), (0,)), ((), ())), preferred_element_type=jnp.float32)
    s = s + bv_ref[...]
    m = jnp.max(s, axis=0, keepdims=True)
    e = jnp.exp(s - m)
    out_ref[...] = (e / jnp.sum(e, axis=0, keepdims=True)).reshape(
        chunk // 128, 128)


def _fused_body(efeat_ref, eids_ref, ewcat_ref, ewv_ref, ebv_ref,
                nfeat_ref, nids_ref, nwcat_ref, nwv_ref, nbv_ref,
                eout_ref, nout_ref, *, echunk, nchunk):
    efeat_t = jnp.transpose(efeat_ref[...].astype(jnp.bfloat16))
    eids = eids_ref[...].reshape(1, echunk)
    eoh = (eids == jax.lax.broadcasted_iota(jnp.int32, (B, echunk), 0)
           ).astype(jnp.bfloat16)
    ea = jnp.concatenate([eoh, efeat_t], axis=0)
    ex = jax.lax.dot_general(
        ewcat_ref[...].astype(jnp.bfloat16), ea,
        (((0,), (0,)), ((), ())), preferred_element_type=jnp.float32)
    et = jnp.tanh(ex).astype(jnp.bfloat16)
    es = jax.lax.dot_general(
        ewv_ref[...].astype(jnp.bfloat16), et,
        (((0,), (0,)), ((), ())), preferred_element_type=jnp.float32)
    es = es + ebv_ref[...]
    em = jnp.max(es, axis=0, keepdims=True)
    ee = jnp.exp(es - em)
    eout_ref[...] = (ee / jnp.sum(ee, axis=0, keepdims=True)).reshape(
        echunk // 128, 128)

    nfeat_t = jnp.transpose(nfeat_ref[...].astype(jnp.bfloat16))
    nids = nids_ref[0]
    noh = (nids == jax.lax.broadcasted_iota(jnp.int32, (B, nchunk), 0)
           ).astype(jnp.bfloat16)
    na = jnp.concatenate([noh, nfeat_t], axis=0)
    nx = jax.lax.dot_general(
        nwcat_ref[...].astype(jnp.bfloat16), na,
        (((0,), (0,)), ((), ())), preferred_element_type=jnp.float32)
    nt = jnp.tanh(nx).astype(jnp.bfloat16)
    ns = jax.lax.dot_general(
        nwv_ref[...].astype(jnp.bfloat16), nt,
        (((0,), (0,)), ((), ())), preferred_element_type=jnp.float32)
    ns = ns + nbv_ref[...]
    nm = jnp.max(ns, axis=0, keepdims=True)
    ne = jnp.exp(ns - nm)
    nout_ref[...] = (ne / jnp.sum(ne, axis=0, keepdims=True))[None]


def kernel(question, node_feat, edge_feat, node_graph_ids, edge_graph_ids,
           Wqn, bqn, Wn, bn, wvn, bvn, Wqe, bqe, We, be, wve, bve):
    qn, qe = pl.pallas_call(
        _qproj_body,
        out_shape=(jax.ShapeDtypeStruct((B, Wqn.shape[1]), jnp.float32),
                   jax.ShapeDtypeStruct((B, Wqe.shape[1]), jnp.float32)),
    )(question, Wqn, bqn.reshape(1, -1), bn.reshape(1, -1),
      Wqe, bqe.reshape(1, -1), be.reshape(1, -1))

    ne_, de = edge_feat.shape
    nn, dn = node_feat.shape
    echunk, nchunk = 25600, 800
    grid = ne_ // echunk
    body = functools.partial(_fused_body, echunk=echunk, nchunk=nchunk)
    eids2 = edge_graph_ids.astype(jnp.int32).reshape(ne_ // 128, 128)
    nids3 = node_graph_ids.astype(jnp.int32).reshape(grid, 1, nchunk)
    eout, nout = pl.pallas_call(
        body,
        grid=(grid,),
        in_specs=[
            pl.BlockSpec((echunk, de), lambda i: (i, 0)),
            pl.BlockSpec((echunk // 128, 128), lambda i: (i, 0)),
            pl.BlockSpec((B + de, 128), lambda i: (0, 0)),
            pl.BlockSpec((128, 1), lambda i: (0, 0)),
            pl.BlockSpec((1, 1), lambda i: (0, 0)),
            pl.BlockSpec((nchunk, dn), lambda i: (i, 0)),
            pl.BlockSpec((1, 1, nchunk), lambda i: (i, 0, 0)),
            pl.BlockSpec((B + dn, 128), lambda i: (0, 0)),
            pl.BlockSpec((128, 1), lambda i: (0, 0)),
            pl.BlockSpec((1, 1), lambda i: (0, 0)),
        ],
        out_specs=(
            pl.BlockSpec((echunk // 128, 128), lambda i: (i, 0)),
            pl.BlockSpec((1, 1, nchunk), lambda i: (i, 0, 0)),
        ),
        out_shape=(
            jax.ShapeDtypeStruct((ne_ // 128, 128), jnp.float32),
            jax.ShapeDtypeStruct((grid, 1, nchunk), jnp.float32),
        ),
        compiler_params=pltpu.CompilerParams(
            dimension_semantics=("parallel",)),
    )(edge_feat, eids2, jnp.concatenate([qe, We]), wve, bve.reshape(1, 1),
      node_feat, nids3, jnp.concatenate([qn, Wn]), wvn, bvn.reshape(1, 1))
    return (nout.reshape(-1), eout.reshape(-1))
